# Initial kernel scaffold; baseline (speedup 1.0000x reference)
import jax
import jax.numpy as jnp
from jax.experimental import pallas as pl


def kernel(segment_ids, embed_table):
    raise NotImplementedError("write your pallas kernel here")


# speedup vs baseline: 1.3866x; 1.3866x over previous
import functools

import jax
import jax.numpy as jnp
from jax import lax
from jax.experimental import pallas as pl
from jax.experimental.pallas import tpu as pltpu
from jax.experimental.pallas import tpu_sc as plsc

B = 16384
L = 200
D = 64
IDXW = 128
NROWS = (B * L) // IDXW
NW = 32
ROWS_PER_W = NROWS // NW
CHUNK = 8
NITER = ROWS_PER_W // CHUNK

_mesh = plsc.VectorSubcoreMesh(core_axis_name="c", subcore_axis_name="s")


@functools.partial(
    pl.kernel,
    mesh=_mesh,
    out_type=jax.ShapeDtypeStruct((NROWS, IDXW, D), jnp.float32),
    scratch_types=[
        pltpu.VMEM((CHUNK, IDXW), jnp.int32),
        pltpu.VMEM((CHUNK, IDXW, D), jnp.float32),
        pltpu.SemaphoreType.DMA,
    ],
    compiler_params=pltpu.CompilerParams(use_tc_tiling_on_sc=False),
)
def _emb_lookup(table_hbm, idx_hbm, out_hbm, idx_v, rows_v, sem):
    wid = lax.axis_index("s") * 2 + lax.axis_index("c")
    base_row = wid * ROWS_PER_W

    def body(g, carry):
        row0 = base_row + g * CHUNK
        pltpu.sync_copy(idx_hbm.at[pl.ds(row0, CHUNK)], idx_v)
        handles = [
            pltpu.async_copy(table_hbm.at[idx_v.at[j]], rows_v.at[j], sem)
            for j in range(CHUNK)
        ]
        for h in handles:
            h.wait()
        pltpu.sync_copy(rows_v, out_hbm.at[pl.ds(row0, CHUNK)])
        return carry

    lax.fori_loop(0, NITER, body, 0)


def kernel(segment_ids, embed_table):
    ids = segment_ids.astype(jnp.int32).reshape(NROWS, IDXW)
    out = _emb_lookup(embed_table, ids)
    return out.reshape(B, L, D)


# speedup vs baseline: 3.5001x; 2.5242x over previous
import functools

import jax
import jax.numpy as jnp
from jax import lax
from jax.experimental import pallas as pl
from jax.experimental.pallas import tpu as pltpu
from jax.experimental.pallas import tpu_sc as plsc

B = 16384
L = 200
D = 64
V = 20
LANES = 16
N_IDX = B * L
NW = 32
IDX_PER_W = N_IDX // NW
CHUNK = 640
PER = CHUNK * D
NITER = IDX_PER_W // CHUNK
NBUF = 2
GROUPS = CHUNK // LANES

_mesh = plsc.VectorSubcoreMesh(core_axis_name="c", subcore_axis_name="s")

_DNUMS = lax.GatherDimensionNumbers(
    offset_dims=(), collapsed_slice_dims=(0,), start_index_map=(0,))


def _lane_bcast(vec, i):
    idx = jnp.full((LANES, 1), i, jnp.int32)
    return lax.gather(vec, idx, _DNUMS, (1,),
                      mode=lax.GatherScatterMode.PROMISE_IN_BOUNDS)


@functools.partial(
    pl.kernel,
    mesh=_mesh,
    out_type=jax.ShapeDtypeStruct((N_IDX * D,), jnp.float32),
    scratch_types=[
        pltpu.VMEM((V, D), jnp.float32),
        pltpu.VMEM((NBUF, CHUNK), jnp.int32),
        pltpu.VMEM((NBUF, PER), jnp.float32),
        pltpu.SemaphoreType.DMA,
        pltpu.SemaphoreType.DMA,
    ],
    compiler_params=pltpu.CompilerParams(needs_layout_passes=False),
)
def _emb_lookup(table_hbm, idx_hbm, out_hbm, table_v, idx_v, rows_v,
                sem_o0, sem_o1):
    sems = [sem_o0, sem_o1]
    wid = lax.axis_index("s") * 2 + lax.axis_index("c")
    base = wid * IDX_PER_W
    pltpu.sync_copy(table_hbm, table_v)
    cols = [jnp.arange(k * LANES, (k + 1) * LANES, dtype=jnp.int32)
            for k in range(D // LANES)]

    def compute_chunk(b, t):
        pltpu.sync_copy(idx_hbm.at[pl.ds(base + t * CHUNK, CHUNK)],
                        idx_v.at[b])

        def group(q, carry):
            ids16 = idx_v[b, pl.ds(q * LANES, LANES)]
            for i in range(LANES):
                rid = _lane_bcast(ids16, i)
                off = (q * LANES + i) * D
                for k in range(D // LANES):
                    vals = plsc.load_gather(table_v, [rid, cols[k]])
                    rows_v[b, pl.ds(off + k * LANES, LANES)] = vals
            return carry

        lax.fori_loop(0, GROUPS, group, 0)

    def out_start(b, t):
        return pltpu.async_copy(
            rows_v.at[b], out_hbm.at[pl.ds((base + t * CHUNK) * D, PER)],
            sems[b])

    def out_wait(b):
        pltpu.make_async_copy(out_hbm.at[pl.ds(0, PER)], rows_v.at[b],
                              sems[b]).wait()

    for b in range(NBUF):
        compute_chunk(b, b)
        out_start(b, b)

    def body(k, carry):
        for b in range(NBUF):
            t = k * NBUF + b
            out_wait(b)
            compute_chunk(b, t)
            out_start(b, t)
        return carry

    lax.fori_loop(1, NITER // NBUF, body, 0)

    for b in range(NBUF):
        out_wait(b)


def kernel(segment_ids, embed_table):
    ids = segment_ids.astype(jnp.int32).reshape(N_IDX)
    out = _emb_lookup(embed_table, ids)
    return out.reshape(B, L, D)


# speedup vs baseline: 5.5847x; 1.5956x over previous
import functools

import jax
import jax.numpy as jnp
from jax import lax
from jax.experimental import pallas as pl
from jax.experimental.pallas import tpu as pltpu
from jax.experimental.pallas import tpu_sc as plsc

B = 16384
L = 200
D = 64
V = 20
IDXW = 128
NROWS = (B * L) // IDXW
NW = 32
ROWS_PER_W = NROWS // NW
CHUNK = 4
NITER = ROWS_PER_W // CHUNK
NBUF = 3
NBLK = (NITER + NBUF - 1) // NBUF

_mesh = plsc.VectorSubcoreMesh(core_axis_name="c", subcore_axis_name="s")


@functools.partial(
    pl.kernel,
    mesh=_mesh,
    out_type=jax.ShapeDtypeStruct((NROWS, IDXW, D), jnp.float32),
    scratch_types=[
        pltpu.VMEM_SHARED((V, D), jnp.float32),
        pltpu.VMEM((NBUF, CHUNK, IDXW), jnp.int32),
        pltpu.VMEM((NBUF, CHUNK, IDXW, D), jnp.float32),
        [pltpu.SemaphoreType.DMA] * NBUF,
        [pltpu.SemaphoreType.DMA] * NBUF,
    ],
    compiler_params=pltpu.CompilerParams(use_tc_tiling_on_sc=False),
)
def _emb_lookup(table_hbm, idx_hbm, out_hbm, table_sh, idx_v, rows_v,
                sem_g, sem_o):
    wid = lax.axis_index("s") * 2 + lax.axis_index("c")
    base = wid * ROWS_PER_W

    @pl.when(lax.axis_index("s") == 0)
    def _stage_table():
        pltpu.sync_copy(table_hbm, table_sh)

    plsc.subcore_barrier()

    def stage_and_gather(b, t):
        row0 = base + t * CHUNK
        pltpu.sync_copy(idx_hbm.at[pl.ds(row0, CHUNK)], idx_v.at[b])
        for j in range(CHUNK):
            pltpu.async_copy(table_sh.at[idx_v.at[b, j]], rows_v.at[b, j],
                             sem_g[b])

    def wait_gathers(b):
        pltpu.make_async_copy(out_hbm.at[pl.ds(0, CHUNK)], rows_v.at[b],
                              sem_g[b]).wait()

    def out_start(b, t):
        pltpu.async_copy(rows_v.at[b],
                         out_hbm.at[pl.ds(base + t * CHUNK, CHUNK)],
                         sem_o[b])

    def out_wait(b):
        pltpu.make_async_copy(out_hbm.at[pl.ds(0, CHUNK)], rows_v.at[b],
                              sem_o[b]).wait()

    for b in range(NBUF):
        stage_and_gather(b, b)

    def body(k, carry):
        for b in range(NBUF):
            t = k * NBUF + b

            @pl.when(t < NITER)
            def _step():
                wait_gathers(b)
                out_start(b, t)
                out_wait(b)

                @pl.when(t + NBUF < NITER)
                def _prefetch():
                    stage_and_gather(b, t + NBUF)

        return carry

    lax.fori_loop(0, NBLK, body, 0)


def kernel(segment_ids, embed_table):
    ids = segment_ids.astype(jnp.int32).reshape(NROWS, IDXW)
    out = _emb_lookup(embed_table, ids)
    return out.reshape(B, L, D)
